# baseline (speedup 1.0000x reference)
import functools

import jax
import jax.numpy as jnp
from jax import lax
from jax.experimental import pallas as pl
from jax.experimental.pallas import tpu as pltpu
from jax.experimental.pallas import tpu_sc as plsc

B = 16384
H = 50
D = 64
NU = 100000
NB = 100000
NCT = 1008

NC = 2
NS = 16
NW = NC * NS
L = 16

RPW = B // NW
NBUF = 4
RPC = 2
CPW = RPW // RPC
IDX_COLS = RPC * H

_mesh = plsc.VectorSubcoreMesh(core_axis_name="c", subcore_axis_name="s")
_SC_PARAMS = pltpu.CompilerParams(use_tc_tiling_on_sc=False,
                                  needs_layout_passes=False)

_GDN = lax.GatherDimensionNumbers(
    offset_dims=(), collapsed_slice_dims=(0,), start_index_map=(0,))


def _bcast_sum(v):
    cs = plsc.cumsum(v)
    idx = jnp.full((L,), L - 1, jnp.int32)
    return lax.gather(cs, idx[:, None], dimension_numbers=_GDN,
                      slice_sizes=(1,),
                      mode=lax.GatherScatterMode.PROMISE_IN_BOUNDS)


def _pack_pairs(y, rows):
    y3 = y.reshape(rows, 2, D)
    return jnp.concatenate([y3[:, 0, :], y3[:, 1, :]], axis=-1)


def _user_p_body(ut_ref, wq_ref, wk_ref, up_ref, pp_ref):
    x = ut_ref[...]
    xt = x.T
    m = lax.dot_general(wq_ref[...], wk_ref[...], (((1,), (1,)), ((), ())),
                        preferred_element_type=jnp.float32) * 0.125
    pt = lax.dot_general(xt, m, (((1,), (0,)), ((), ())),
                         preferred_element_type=jnp.float32)
    up_ref[...] = _pack_pairs(xt, xt.shape[0] // 2)
    pp_ref[...] = _pack_pairs(pt, pt.shape[0] // 2)


def _biz_body(bt_ref, bp_ref):
    xt = bt_ref[...].T
    bp_ref[...] = _pack_pairs(xt, xt.shape[0] // 2)


_BC = 2560
_NGRID = 40
NROWS = _BC * _NGRID

_user_p = pl.pallas_call(
    _user_p_body,
    grid=(_NGRID,),
    in_specs=[
        pl.BlockSpec((D, _BC), lambda g: (0, g)),
        pl.BlockSpec((D, D), lambda g: (0, 0)),
        pl.BlockSpec((D, D), lambda g: (0, 0)),
    ],
    out_specs=[
        pl.BlockSpec((_BC // 2, 128), lambda g: (g, 0)),
        pl.BlockSpec((_BC // 2, 128), lambda g: (g, 0)),
    ],
    out_shape=[
        jax.ShapeDtypeStruct((NROWS // 2, 128), jnp.float32),
        jax.ShapeDtypeStruct((NROWS // 2, 128), jnp.float32),
    ],
)

_biz = pl.pallas_call(
    _biz_body,
    grid=(_NGRID,),
    in_specs=[pl.BlockSpec((D, _BC), lambda g: (0, g))],
    out_specs=pl.BlockSpec((_BC // 2, 128), lambda g: (g, 0)),
    out_shape=jax.ShapeDtypeStruct((NROWS // 2, 128), jnp.float32),
)


def _small_body(ct_ref, gt_ref, cp_ref, gp_ref):
    zpad = jnp.zeros((NCT - 1001, D), jnp.float32)

    def packed(ref):
        xt = ref[...].T
        return _pack_pairs(jnp.concatenate([xt, zpad], axis=0), NCT // 2)

    cp_ref[...] = packed(ct_ref)
    gp_ref[...] = packed(gt_ref)


_small = pl.pallas_call(
    _small_body,
    grid=(1,),
    in_specs=[
        pl.BlockSpec((D, 1001), lambda g: (0, 0)),
        pl.BlockSpec((D, 1001), lambda g: (0, 0)),
    ],
    out_specs=[
        pl.BlockSpec((NCT // 2, 128), lambda g: (0, 0)),
        pl.BlockSpec((NCT // 2, 128), lambda g: (0, 0)),
    ],
    out_shape=[
        jax.ShapeDtypeStruct((NCT // 2, 128), jnp.float32),
        jax.ShapeDtypeStruct((NCT // 2, 128), jnp.float32),
    ],
)


@functools.partial(
    pl.kernel,
    out_type=(
        jax.ShapeDtypeStruct((B, D), jnp.float32),
        jax.ShapeDtypeStruct((B, D), jnp.float32),
    ),
    mesh=_mesh,
    scratch_types=[
        pltpu.VMEM((4, 128), jnp.int32),
        pltpu.VMEM((CPW, IDX_COLS), jnp.int32),
        pltpu.VMEM((RPW, D), jnp.float32),
        pltpu.VMEM((NBUF, IDX_COLS, D), jnp.float32),
        pltpu.VMEM((RPW, D), jnp.float32),
        pltpu.SemaphoreType.DMA((4,)),
    ],
    compiler_params=_SC_PARAMS,
)
def _mega_kernel(utab, ctab, gtab, btab, ptab, uidx, cidx, gidx, bidx,
                 s_out, a_out, ib, idxv, pbuf, gbuf, obuf, sems):
    wid = lax.axis_index("s") * NC + lax.axis_index("c")
    base = wid * RPW

    def gather512(tab, idx2, dst):
        pltpu.sync_copy(idx2.at[pl.ds(wid * 4, 4)], ib)
        for j in range(4):
            pltpu.async_copy(tab.at[ib.at[j]], dst.at[pl.ds(128 * j, 128)],
                             sems.at[j])
        for j in range(4):
            pltpu.make_async_copy(tab.at[ib.at[j]],
                                  dst.at[pl.ds(128 * j, 128)],
                                  sems.at[j]).wait()

    def accumulate():
        def body(r, _):
            for j in range(4):
                sl = pl.ds(L * j, L)
                pbuf[r, sl] = pbuf[r, sl] + obuf[r, sl]
            return ()
        lax.fori_loop(0, RPW, body, ())

    gather512(utab, uidx, pbuf)
    gather512(ctab, cidx, obuf)
    accumulate()
    gather512(gtab, gidx, obuf)
    accumulate()
    pltpu.sync_copy(pbuf, s_out.at[pl.ds(base, RPW)])

    gather512(ptab, uidx, pbuf)
    pltpu.sync_copy(bidx.at[pl.ds(wid * CPW, CPW)], idxv)

    def start(c, k):
        pltpu.async_copy(btab.at[idxv.at[c]], gbuf.at[k], sems.at[k])

    def wait(c, k):
        pltpu.make_async_copy(btab.at[idxv.at[c]], gbuf.at[k],
                              sems.at[k]).wait()

    def compute(c, k):
        for r in range(RPC):
            row = RPC * c + r
            pv = tuple(pbuf[row, pl.ds(L * j, L)] for j in range(4))
            zero = jnp.zeros((L,), jnp.float32)

            def hbody(it, carry):
                a0, a1, a2, a3, den = carry
                for t in range(10):
                    hrow = r * H + it * 10 + t
                    b0 = gbuf[k, hrow, pl.ds(0, L)]
                    b1 = gbuf[k, hrow, pl.ds(L, L)]
                    b2 = gbuf[k, hrow, pl.ds(2 * L, L)]
                    b3 = gbuf[k, hrow, pl.ds(3 * L, L)]
                    dv = (pv[0] * b0 + pv[1] * b1) + (pv[2] * b2 + pv[3] * b3)
                    e = jnp.exp(_bcast_sum(dv))
                    den = den + e
                    a0 = a0 + e * b0
                    a1 = a1 + e * b1
                    a2 = a2 + e * b2
                    a3 = a3 + e * b3
                return a0, a1, a2, a3, den

            a0, a1, a2, a3, den = lax.fori_loop(
                0, H // 10, hbody, (zero, zero, zero, zero, zero))
            r_den = 1.0 / den
            obuf[row, pl.ds(0, L)] = a0 * r_den
            obuf[row, pl.ds(L, L)] = a1 * r_den
            obuf[row, pl.ds(2 * L, L)] = a2 * r_den
            obuf[row, pl.ds(3 * L, L)] = a3 * r_den

    for k in range(NBUF):
        start(k, k)

    def loop_body(i, _):
        for k in range(NBUF):
            c = i * NBUF + k
            @pl.when(i == 0)
            def _():
                wait(c, k)
            compute(c, k)
        return ()

    lax.fori_loop(0, CPW // NBUF, loop_body, ())
    pltpu.sync_copy(obuf, a_out.at[pl.ds(base, RPW)])


def _final_body(s_ref, a_ref, wv_ref, o_ref):
    wv = wv_ref[...]
    lo = s_ref[:, :D] + lax.dot_general(
        a_ref[:, :D], wv, (((1,), (0,)), ((), ())),
        preferred_element_type=jnp.float32)
    hi = s_ref[:, D:] + lax.dot_general(
        a_ref[:, D:], wv, (((1,), (0,)), ((), ())),
        preferred_element_type=jnp.float32)
    rows = jnp.concatenate([lo[:, None, :], hi[:, None, :]], axis=1)
    o_ref[...] = rows.reshape(2 * lo.shape[0], D).T


_FB = 512

_final = pl.pallas_call(
    _final_body,
    grid=(B // 2 // _FB,),
    in_specs=[
        pl.BlockSpec((_FB, 128), lambda g: (g, 0)),
        pl.BlockSpec((_FB, 128), lambda g: (g, 0)),
        pl.BlockSpec((D, D), lambda g: (0, 0)),
    ],
    out_specs=pl.BlockSpec((D, 2 * _FB), lambda g: (0, g)),
    out_shape=jax.ShapeDtypeStruct((D, B), jnp.float32))


def kernel(user_table, business_table, city_table, category_table,
           Wq, Wk, Wv, user_idx, business_neigh_idx, city_idx, category_idx):
    uidx = user_idx.astype(jnp.int32).reshape(B // 128, 128)
    cidx = city_idx.astype(jnp.int32).reshape(B // 128, 128)
    gidx = category_idx.astype(jnp.int32).reshape(B // 128, 128)
    bidx = business_neigh_idx.astype(jnp.int32).reshape(B // RPC, IDX_COLS)

    up, pp = _user_p(user_table.T, Wq, Wk)
    bp = _biz(business_table.T)
    cp, gp = _small(city_table.T, category_table.T)

    s, anorm = _mega_kernel(
        up.reshape(NROWS, D), cp.reshape(NCT, D), gp.reshape(NCT, D),
        bp.reshape(NROWS, D), pp.reshape(NROWS, D),
        uidx, cidx, gidx, bidx)

    out_t = _final(s.reshape(B // 2, 128), anorm.reshape(B // 2, 128), Wv)
    return out_t.T


# speedup vs baseline: 1.0110x; 1.0110x over previous
import functools

import jax
import jax.numpy as jnp
from jax import lax
from jax.experimental import pallas as pl
from jax.experimental.pallas import tpu as pltpu
from jax.experimental.pallas import tpu_sc as plsc

B = 16384
H = 50
D = 64
NU = 100000
NB = 100000
NCT = 1008

NC = 2
NS = 16
NW = NC * NS
L = 16

RPW = B // NW
NBUF = 4
RPC = 2
CPW = RPW // RPC
IDX_COLS = RPC * H

_mesh = plsc.VectorSubcoreMesh(core_axis_name="c", subcore_axis_name="s")
_SC_PARAMS = pltpu.CompilerParams(use_tc_tiling_on_sc=False,
                                  needs_layout_passes=False)

_GDN = lax.GatherDimensionNumbers(
    offset_dims=(), collapsed_slice_dims=(0,), start_index_map=(0,))


def _bcast_sum(v):
    cs = plsc.cumsum(v)
    idx = jnp.full((L,), L - 1, jnp.int32)
    return lax.gather(cs, idx[:, None], dimension_numbers=_GDN,
                      slice_sizes=(1,),
                      mode=lax.GatherScatterMode.PROMISE_IN_BOUNDS)


def _pack_pairs(y, rows):
    y3 = y.reshape(rows, 2, D)
    return jnp.concatenate([y3[:, 0, :], y3[:, 1, :]], axis=-1)


def _user_p_body(ut_ref, wq_ref, wk_ref, up_ref, pp_ref):
    x = ut_ref[...]
    xt = x.T
    m = lax.dot_general(wq_ref[...], wk_ref[...], (((1,), (1,)), ((), ())),
                        preferred_element_type=jnp.float32) * 0.125
    pt = lax.dot_general(xt, m, (((1,), (0,)), ((), ())),
                         preferred_element_type=jnp.float32)
    up_ref[...] = _pack_pairs(xt, xt.shape[0] // 2)
    pp_ref[...] = _pack_pairs(pt, pt.shape[0] // 2)


def _biz_body(bt_ref, bp_ref):
    xt = bt_ref[...].T
    bp_ref[...] = _pack_pairs(xt, xt.shape[0] // 2)


_BC = 2560
_NGRID = 40
NROWS = _BC * _NGRID

_user_p = pl.pallas_call(
    _user_p_body,
    grid=(_NGRID,),
    in_specs=[
        pl.BlockSpec((D, _BC), lambda g: (0, g)),
        pl.BlockSpec((D, D), lambda g: (0, 0)),
        pl.BlockSpec((D, D), lambda g: (0, 0)),
    ],
    out_specs=[
        pl.BlockSpec((_BC // 2, 128), lambda g: (g, 0)),
        pl.BlockSpec((_BC // 2, 128), lambda g: (g, 0)),
    ],
    out_shape=[
        jax.ShapeDtypeStruct((NROWS // 2, 128), jnp.float32),
        jax.ShapeDtypeStruct((NROWS // 2, 128), jnp.float32),
    ],
)

_biz = pl.pallas_call(
    _biz_body,
    grid=(_NGRID,),
    in_specs=[pl.BlockSpec((D, _BC), lambda g: (0, g))],
    out_specs=pl.BlockSpec((_BC // 2, 128), lambda g: (g, 0)),
    out_shape=jax.ShapeDtypeStruct((NROWS // 2, 128), jnp.float32),
)


def _small_body(ct_ref, gt_ref, cp_ref, gp_ref):
    zpad = jnp.zeros((NCT - 1001, D), jnp.float32)

    def packed(ref):
        xt = ref[...].T
        return _pack_pairs(jnp.concatenate([xt, zpad], axis=0), NCT // 2)

    cp_ref[...] = packed(ct_ref)
    gp_ref[...] = packed(gt_ref)


_small = pl.pallas_call(
    _small_body,
    grid=(1,),
    in_specs=[
        pl.BlockSpec((D, 1001), lambda g: (0, 0)),
        pl.BlockSpec((D, 1001), lambda g: (0, 0)),
    ],
    out_specs=[
        pl.BlockSpec((NCT // 2, 128), lambda g: (0, 0)),
        pl.BlockSpec((NCT // 2, 128), lambda g: (0, 0)),
    ],
    out_shape=[
        jax.ShapeDtypeStruct((NCT // 2, 128), jnp.float32),
        jax.ShapeDtypeStruct((NCT // 2, 128), jnp.float32),
    ],
)


@functools.partial(
    pl.kernel,
    out_type=(
        jax.ShapeDtypeStruct((B, D), jnp.float32),
        jax.ShapeDtypeStruct((B, D), jnp.float32),
    ),
    mesh=_mesh,
    scratch_types=[
        pltpu.VMEM((4, 128), jnp.int32),
        pltpu.VMEM((CPW, IDX_COLS), jnp.int32),
        pltpu.VMEM((RPW, D), jnp.float32),
        pltpu.VMEM((NBUF, IDX_COLS, D), jnp.float32),
        pltpu.VMEM((RPW, D), jnp.float32),
        pltpu.SemaphoreType.DMA((4,)),
    ],
    compiler_params=_SC_PARAMS,
)
def _mega_kernel(utab, ctab, gtab, btab, ptab, uidx, cidx, gidx, bidx,
                 s_out, a_out, ib, idxv, pbuf, gbuf, obuf, sems):
    wid = lax.axis_index("s") * NC + lax.axis_index("c")
    base = wid * RPW

    def gather512(tab, idx2, dst):
        pltpu.sync_copy(idx2.at[pl.ds(wid * 4, 4)], ib)
        for j in range(4):
            pltpu.async_copy(tab.at[ib.at[j]], dst.at[pl.ds(128 * j, 128)],
                             sems.at[j])
        for j in range(4):
            pltpu.make_async_copy(tab.at[ib.at[j]],
                                  dst.at[pl.ds(128 * j, 128)],
                                  sems.at[j]).wait()

    def accumulate():
        def body(r, _):
            for j in range(4):
                sl = pl.ds(L * j, L)
                pbuf[r, sl] = pbuf[r, sl] + obuf[r, sl]
            return ()
        lax.fori_loop(0, RPW, body, ())

    gather512(utab, uidx, pbuf)
    gather512(ctab, cidx, obuf)
    accumulate()
    gather512(gtab, gidx, obuf)
    accumulate()
    pltpu.sync_copy(pbuf, s_out.at[pl.ds(base, RPW)])

    gather512(ptab, uidx, pbuf)
    pltpu.sync_copy(bidx.at[pl.ds(wid * CPW, CPW)], idxv)

    def start(c, k):
        pltpu.async_copy(btab.at[idxv.at[c]], gbuf.at[k], sems.at[k])

    def wait(c, k):
        pltpu.make_async_copy(btab.at[idxv.at[c]], gbuf.at[k],
                              sems.at[k]).wait()

    def compute(c, k):
        for r in range(RPC):
            row = RPC * c + r
            pv = tuple(pbuf[row, pl.ds(L * j, L)] for j in range(4))
            zero = jnp.zeros((L,), jnp.float32)

            @plsc.parallel_loop(0, H, step=5, unroll=1,
                                carry=(zero, zero, zero, zero, zero))
            def hloop(h0, acc):
                a0, a1, a2, a3, den = acc
                for t in range(5):
                    hrow = r * H + h0 + t
                    b0 = gbuf[k, hrow, pl.ds(0, L)]
                    b1 = gbuf[k, hrow, pl.ds(L, L)]
                    b2 = gbuf[k, hrow, pl.ds(2 * L, L)]
                    b3 = gbuf[k, hrow, pl.ds(3 * L, L)]
                    dv = (pv[0] * b0 + pv[1] * b1) + (pv[2] * b2 + pv[3] * b3)
                    e = jnp.exp(_bcast_sum(dv))
                    den = den + e
                    a0 = a0 + e * b0
                    a1 = a1 + e * b1
                    a2 = a2 + e * b2
                    a3 = a3 + e * b3
                return a0, a1, a2, a3, den

            a0, a1, a2, a3, den = hloop
            r_den = 1.0 / den
            obuf[row, pl.ds(0, L)] = a0 * r_den
            obuf[row, pl.ds(L, L)] = a1 * r_den
            obuf[row, pl.ds(2 * L, L)] = a2 * r_den
            obuf[row, pl.ds(3 * L, L)] = a3 * r_den

    for k in range(NBUF):
        start(k, k)

    def loop_body(i, _):
        for k in range(NBUF):
            c = i * NBUF + k
            wait(c, k)
            compute(c, k)

            @pl.when(i < CPW // NBUF - 1)
            def _():
                start(c + NBUF, k)
        return ()

    lax.fori_loop(0, CPW // NBUF, loop_body, ())
    pltpu.sync_copy(obuf, a_out.at[pl.ds(base, RPW)])


def _final_body(s_ref, a_ref, wv_ref, o_ref):
    wv = wv_ref[...]
    lo = s_ref[:, :D] + lax.dot_general(
        a_ref[:, :D], wv, (((1,), (0,)), ((), ())),
        preferred_element_type=jnp.float32)
    hi = s_ref[:, D:] + lax.dot_general(
        a_ref[:, D:], wv, (((1,), (0,)), ((), ())),
        preferred_element_type=jnp.float32)
    rows = jnp.concatenate([lo[:, None, :], hi[:, None, :]], axis=1)
    o_ref[...] = rows.reshape(2 * lo.shape[0], D).T


_FB = 512

_final = pl.pallas_call(
    _final_body,
    grid=(B // 2 // _FB,),
    in_specs=[
        pl.BlockSpec((_FB, 128), lambda g: (g, 0)),
        pl.BlockSpec((_FB, 128), lambda g: (g, 0)),
        pl.BlockSpec((D, D), lambda g: (0, 0)),
    ],
    out_specs=pl.BlockSpec((D, 2 * _FB), lambda g: (0, g)),
    out_shape=jax.ShapeDtypeStruct((D, B), jnp.float32))


def kernel(user_table, business_table, city_table, category_table,
           Wq, Wk, Wv, user_idx, business_neigh_idx, city_idx, category_idx):
    uidx = user_idx.astype(jnp.int32).reshape(B // 128, 128)
    cidx = city_idx.astype(jnp.int32).reshape(B // 128, 128)
    gidx = category_idx.astype(jnp.int32).reshape(B // 128, 128)
    bidx = business_neigh_idx.astype(jnp.int32).reshape(B // RPC, IDX_COLS)

    up, pp = _user_p(user_table.T, Wq, Wk)
    bp = _biz(business_table.T)
    cp, gp = _small(city_table.T, category_table.T)

    s, anorm = _mega_kernel(
        up.reshape(NROWS, D), cp.reshape(NCT, D), gp.reshape(NCT, D),
        bp.reshape(NROWS, D), pp.reshape(NROWS, D),
        uidx, cidx, gidx, bidx)

    out_t = _final(s.reshape(B // 2, 128), anorm.reshape(B // 2, 128), Wv)
    return out_t.T
